# baseline (speedup 1.0000x reference)
import functools

import jax
from jax import lax
from jax.experimental import pallas as pl
from jax.experimental.pallas import tpu as pltpu
from jax.experimental.pallas import tpu_sc as plsc

_CHUNK = 16
_NBUF = 3
_PRE = 2


def _tc_fast_copy(frames):
    C, T, H, W = frames.shape
    nchunks = T // _CHUNK

    def body(x_ref, fast_ref, buf, in_sem, out_sem):
        def in_copy(k):
            return pltpu.make_async_copy(
                x_ref.at[:, k * _CHUNK:(k + 1) * _CHUNK],
                buf.at[k % _NBUF], in_sem.at[k % _NBUF])

        def out_copies(k):
            h = _CHUNK // 2
            return [
                pltpu.make_async_copy(
                    buf.at[k % _NBUF, :, 0:h],
                    fast_ref.at[:, k * _CHUNK:k * _CHUNK + h],
                    out_sem.at[k % _NBUF, 0]),
                pltpu.make_async_copy(
                    buf.at[k % _NBUF, :, h:_CHUNK],
                    fast_ref.at[:, k * _CHUNK + h:(k + 1) * _CHUNK],
                    out_sem.at[k % _NBUF, 1]),
            ]

        for k in range(_PRE):
            in_copy(k).start()
        for k in range(nchunks):
            look = k + _PRE
            if look < nchunks:
                prev = look - _NBUF
                if prev >= 0:
                    for c in out_copies(prev):
                        c.wait()
                in_copy(look).start()
            in_copy(k).wait()
            for c in out_copies(k):
                c.start()
        for k in range(max(0, nchunks - _NBUF), nchunks):
            for c in out_copies(k):
                c.wait()

    return pl.pallas_call(
        body,
        in_specs=[pl.BlockSpec(memory_space=pl.ANY)],
        out_specs=pl.BlockSpec(memory_space=pl.ANY),
        out_shape=jax.ShapeDtypeStruct((C, T, H, W), frames.dtype),
        scratch_shapes=[
            pltpu.VMEM((_NBUF, C, _CHUNK, H, W), frames.dtype),
            pltpu.SemaphoreType.DMA((_NBUF,)),
            pltpu.SemaphoreType.DMA((_NBUF, 2)),
        ],
    )(frames)


def _sc_gather(frames):
    C, T, H, W = frames.shape
    S = T // 4
    info = plsc.get_sparse_core_info()
    NC, NS = info.num_cores, info.num_subcores
    NW = NC * NS
    nrows = C * S

    mesh = plsc.VectorSubcoreMesh(core_axis_name="c", subcore_axis_name="s")

    @functools.partial(
        pl.kernel,
        out_type=jax.ShapeDtypeStruct((C, S, H, W), frames.dtype),
        mesh=mesh,
        scratch_types=[pltpu.SemaphoreType.DMA],
    )
    def gather(x, slow, sem):
        w = lax.axis_index("s") * NC + lax.axis_index("c")

        def copy_j(j):
            c = j // S
            jj = j % S
            t = jj * (T - 1) // (S - 1)
            return pltpu.make_async_copy(x.at[c, t], slow.at[c, jj], sem)

        copy_j(w).start()

        @pl.when(w < nrows - NW)
        def _():
            copy_j(w + NW).start()
            copy_j(w + NW).wait()

        copy_j(w).wait()

    return gather(frames)


def kernel(frames):
    return (_sc_gather(frames), _tc_fast_copy(frames))


# speedup vs baseline: 6.9467x; 6.9467x over previous
import functools

import jax
from jax import lax
from jax.experimental import pallas as pl
from jax.experimental.pallas import tpu as pltpu
from jax.experimental.pallas import tpu_sc as plsc

_CHUNK = 16
_NBUF = 3
_PRE = 2


def _tc_fast_copy(frames):
    C, T, H, W = frames.shape
    nchunks = T // _CHUNK

    def body(x_ref, fast_ref, buf, in_sem, out_sem):
        def in_copy(k):
            return pltpu.make_async_copy(
                x_ref.at[:, k * _CHUNK:(k + 1) * _CHUNK],
                buf.at[k % _NBUF], in_sem.at[k % _NBUF])

        def out_copies(k):
            h = _CHUNK // 2
            return [
                pltpu.make_async_copy(
                    buf.at[k % _NBUF, :, 0:h],
                    fast_ref.at[:, k * _CHUNK:k * _CHUNK + h],
                    out_sem.at[k % _NBUF, 0]),
                pltpu.make_async_copy(
                    buf.at[k % _NBUF, :, h:_CHUNK],
                    fast_ref.at[:, k * _CHUNK + h:(k + 1) * _CHUNK],
                    out_sem.at[k % _NBUF, 1]),
            ]

        for k in range(_PRE):
            in_copy(k).start()
        for k in range(nchunks):
            look = k + _PRE
            if look < nchunks:
                prev = look - _NBUF
                if prev >= 0:
                    for c in out_copies(prev):
                        c.wait()
                in_copy(look).start()
            in_copy(k).wait()
            for c in out_copies(k):
                c.start()
        for k in range(max(0, nchunks - _NBUF), nchunks):
            for c in out_copies(k):
                c.wait()

    return pl.pallas_call(
        body,
        in_specs=[pl.BlockSpec(memory_space=pl.ANY)],
        out_specs=pl.BlockSpec(memory_space=pl.ANY),
        out_shape=jax.ShapeDtypeStruct((C, T, H, W), frames.dtype),
        scratch_shapes=[
            pltpu.VMEM((_NBUF, C, _CHUNK, H, W), frames.dtype),
            pltpu.SemaphoreType.DMA((_NBUF,)),
            pltpu.SemaphoreType.DMA((_NBUF, 2)),
        ],
    )(frames)


def _sc_gather(frames):
    C, T, H, W = frames.shape
    S = T // 4
    info = plsc.get_sparse_core_info()
    NC, NS = info.num_cores, info.num_subcores
    NW = NC * NS
    nrows = C * S

    mesh = plsc.VectorSubcoreMesh(core_axis_name="c", subcore_axis_name="s")

    @functools.partial(
        pl.kernel,
        out_type=jax.ShapeDtypeStruct((C, S, H, W), frames.dtype),
        mesh=mesh,
        scratch_types=[
            pltpu.VMEM((2, H, W), frames.dtype),
            pltpu.SemaphoreType.DMA((2,)),
            pltpu.SemaphoreType.DMA((2,)),
        ],
    )
    def gather(x, slow, buf, in_sem, out_sem):
        w = lax.axis_index("s") * NC + lax.axis_index("c")

        def in_copy(j, b):
            c = j // S
            jj = j % S
            t = jj * (T - 1) // (S - 1)
            return pltpu.make_async_copy(x.at[c, t], buf.at[b], in_sem.at[b])

        def out_copy(j, b):
            c = j // S
            jj = j % S
            return pltpu.make_async_copy(buf.at[b], slow.at[c, jj],
                                         out_sem.at[b])

        second = w < nrows - NW
        in_copy(w, 0).start()

        @pl.when(second)
        def _():
            in_copy(w + NW, 1).start()

        in_copy(w, 0).wait()
        out_copy(w, 0).start()

        @pl.when(second)
        def _():
            in_copy(w + NW, 1).wait()
            out_copy(w + NW, 1).start()
            out_copy(w + NW, 1).wait()

        out_copy(w, 0).wait()

    return gather(frames)


def kernel(frames):
    return (_sc_gather(frames), _tc_fast_copy(frames))


# speedup vs baseline: 6.9595x; 1.0018x over previous
import functools

import jax
from jax import lax
from jax.experimental import pallas as pl
from jax.experimental.pallas import tpu as pltpu
from jax.experimental.pallas import tpu_sc as plsc

_CHUNK = 16
_NBUF = 3
_PRE = 2
_QSPLIT = 4


def _tc_fast_copy(frames):
    C, T, H, W = frames.shape
    nchunks = T // _CHUNK

    def body(x_ref, fast_ref, buf, in_sem, out_sem):
        def in_copy(k):
            return pltpu.make_async_copy(
                x_ref.at[:, k * _CHUNK:(k + 1) * _CHUNK],
                buf.at[k % _NBUF], in_sem.at[k % _NBUF])

        def out_copies(k):
            h = _CHUNK // 2
            return [
                pltpu.make_async_copy(
                    buf.at[k % _NBUF, :, 0:h],
                    fast_ref.at[:, k * _CHUNK:k * _CHUNK + h],
                    out_sem.at[k % _NBUF, 0]),
                pltpu.make_async_copy(
                    buf.at[k % _NBUF, :, h:_CHUNK],
                    fast_ref.at[:, k * _CHUNK + h:(k + 1) * _CHUNK],
                    out_sem.at[k % _NBUF, 1]),
            ]

        for k in range(_PRE):
            in_copy(k).start()
        for k in range(nchunks):
            look = k + _PRE
            if look < nchunks:
                prev = look - _NBUF
                if prev >= 0:
                    for c in out_copies(prev):
                        c.wait()
                in_copy(look).start()
            in_copy(k).wait()
            for c in out_copies(k):
                c.start()
        for k in range(max(0, nchunks - _NBUF), nchunks):
            for c in out_copies(k):
                c.wait()

    return pl.pallas_call(
        body,
        in_specs=[pl.BlockSpec(memory_space=pl.ANY)],
        out_specs=pl.BlockSpec(memory_space=pl.ANY),
        out_shape=jax.ShapeDtypeStruct((C, T, H, W), frames.dtype),
        scratch_shapes=[
            pltpu.VMEM((_NBUF, C, _CHUNK, H, W), frames.dtype),
            pltpu.SemaphoreType.DMA((_NBUF,)),
            pltpu.SemaphoreType.DMA((_NBUF, 2)),
        ],
    )(frames)


def _sc_gather(frames):
    C, T, H, W = frames.shape
    S = T // 4
    info = plsc.get_sparse_core_info()
    NC, NS = info.num_cores, info.num_subcores
    NW = NC * NS
    nrows = C * S

    mesh = plsc.VectorSubcoreMesh(core_axis_name="c", subcore_axis_name="s")

    QS = _QSPLIT
    HQ = H // QS
    tpw = (nrows * QS) // NW
    nbuf = 4
    pre = 3

    @functools.partial(
        pl.kernel,
        out_type=jax.ShapeDtypeStruct((C, S, H, W), frames.dtype),
        mesh=mesh,
        scratch_types=[
            pltpu.VMEM((nbuf, HQ, W), frames.dtype),
            pltpu.SemaphoreType.DMA((nbuf,)),
            pltpu.SemaphoreType.DMA((nbuf,)),
        ],
    )
    def gather(x, slow, buf, in_sem, out_sem):
        w = lax.axis_index("s") * NC + lax.axis_index("c")

        def task(i):
            tau = w * tpw + i
            j = tau // QS
            q = tau % QS
            c = j // S
            jj = j % S
            t = jj * (T - 1) // (S - 1)
            return c, t, jj, q

        def in_copy(i, b):
            c, t, jj, q = task(i)
            return pltpu.make_async_copy(
                x.at[c, t, pl.ds(q * HQ, HQ)], buf.at[b], in_sem.at[b])

        def out_copy(i, b):
            c, t, jj, q = task(i)
            return pltpu.make_async_copy(
                buf.at[b], slow.at[c, jj, pl.ds(q * HQ, HQ)], out_sem.at[b])

        for i in range(pre):
            in_copy(i, i % nbuf).start()
        for i in range(tpw):
            look = i + pre
            if look < tpw:
                prev = look - nbuf
                if prev >= 0:
                    out_copy(prev, prev % nbuf).wait()
                in_copy(look, look % nbuf).start()
            in_copy(i, i % nbuf).wait()
            out_copy(i, i % nbuf).start()
        for i in range(max(0, tpw - nbuf), tpw):
            out_copy(i, i % nbuf).wait()

    return gather(frames)


def kernel(frames):
    return (_sc_gather(frames), _tc_fast_copy(frames))


# speedup vs baseline: 11.6449x; 1.6732x over previous
import numpy as np
import jax
from jax.experimental import pallas as pl
from jax.experimental.pallas import tpu as pltpu

_CHUNK = 16
_NBUF = 3
_PRE = 2


def kernel(frames):
    C, T, H, W = frames.shape
    S = T // 4
    idx = [int(v) for v in np.linspace(0, T - 1, S).astype(np.int64)]
    nchunks = T // _CHUNK
    chunk_gather = [
        [(t - k * _CHUNK, j) for j, t in enumerate(idx)
         if k * _CHUNK <= t < (k + 1) * _CHUNK]
        for k in range(nchunks)
    ]

    def body(x_ref, slow_ref, fast_ref, buf, in_sem, out_sem):
        def in_copy(k):
            return pltpu.make_async_copy(
                x_ref.at[:, k * _CHUNK:(k + 1) * _CHUNK],
                buf.at[k % _NBUF], in_sem.at[k % _NBUF])

        def out_copies(k):
            h = _CHUNK // 2
            cs = [
                pltpu.make_async_copy(
                    buf.at[k % _NBUF, :, 0:h],
                    fast_ref.at[:, k * _CHUNK:k * _CHUNK + h],
                    out_sem.at[k % _NBUF, 0]),
                pltpu.make_async_copy(
                    buf.at[k % _NBUF, :, h:_CHUNK],
                    fast_ref.at[:, k * _CHUNK + h:(k + 1) * _CHUNK],
                    out_sem.at[k % _NBUF, 1]),
            ]
            for off, j in chunk_gather[k]:
                cs.append(pltpu.make_async_copy(
                    buf.at[k % _NBUF, :, off:off + 1],
                    slow_ref.at[:, j:j + 1], out_sem.at[k % _NBUF, 2]))
            return cs

        for k in range(_PRE):
            in_copy(k).start()
        for k in range(nchunks):
            look = k + _PRE
            if look < nchunks:
                prev = look - _NBUF
                if prev >= 0:
                    for c in out_copies(prev):
                        c.wait()
                in_copy(look).start()
            in_copy(k).wait()
            for c in out_copies(k):
                c.start()
        for k in range(max(0, nchunks - _NBUF), nchunks):
            for c in out_copies(k):
                c.wait()

    slow, fast = pl.pallas_call(
        body,
        in_specs=[pl.BlockSpec(memory_space=pl.ANY)],
        out_specs=[
            pl.BlockSpec(memory_space=pl.ANY),
            pl.BlockSpec(memory_space=pl.ANY),
        ],
        out_shape=[
            jax.ShapeDtypeStruct((C, S, H, W), frames.dtype),
            jax.ShapeDtypeStruct((C, T, H, W), frames.dtype),
        ],
        scratch_shapes=[
            pltpu.VMEM((_NBUF, C, _CHUNK, H, W), frames.dtype),
            pltpu.SemaphoreType.DMA((_NBUF,)),
            pltpu.SemaphoreType.DMA((_NBUF, 3)),
        ],
    )(frames)
    return (slow, fast)
